# baseline (speedup 1.0000x reference)
import jax
import jax.numpy as jnp
from jax.experimental import pallas as pl
from jax.experimental.pallas import tpu as pltpu

D_MODEL = 2048
NUM_EXPERTS = 64
MAX_SEQ_LEN = 4096
SPREAD = 2.0

TILE = 512


def _router_kernel(x_ref, sig_ref, idx_ref, soft_ref):
    i = pl.program_id(0)
    x = x_ref[...]
    sigs = jnp.sign(sig_ref[...])
    scores = jax.lax.dot_general(
        x, sigs, (((1,), (1,)), ((), ())),
        preferred_element_type=jnp.float32)

    t0 = (i * TILE) % MAX_SEQ_LEN
    t = t0 + jax.lax.broadcasted_iota(
        jnp.int32, (TILE, NUM_EXPERTS), 0).astype(jnp.float32)
    centers = jax.lax.broadcasted_iota(
        jnp.int32, (TILE, NUM_EXPERTS), 1).astype(jnp.float32)
    u = (t * (NUM_EXPERTS / MAX_SEQ_LEN) - centers) * (1.0 / SPREAD)
    a = jnp.abs(u)
    pos = jnp.where(
        a < 1.0, 2.0 / 3.0 - a * a + 0.5 * a * a * a,
        jnp.where(a < 2.0, (1.0 / 6.0) * (2.0 - a) ** 3, 0.0))

    combined = scores * pos

    m = jnp.max(combined, axis=1, keepdims=True)
    e = jnp.exp(combined - m)
    s = jnp.sum(e, axis=1, keepdims=True)
    soft_ref[...] = e / s

    lane = jax.lax.broadcasted_iota(jnp.int32, (TILE, NUM_EXPERTS), 1)
    cand = jnp.where(combined == m, lane, NUM_EXPERTS)
    idx_ref[...] = jnp.min(cand, axis=1, keepdims=True)


def kernel(x, signatures):
    B, T, D = x.shape
    M = B * T
    xf = x.reshape(M, D)
    grid = (M // TILE,)
    idx, soft = pl.pallas_call(
        _router_kernel,
        grid=grid,
        in_specs=[
            pl.BlockSpec((TILE, D), lambda i: (i, 0)),
            pl.BlockSpec((NUM_EXPERTS, D), lambda i: (0, 0)),
        ],
        out_specs=[
            pl.BlockSpec((TILE, 1), lambda i: (i, 0)),
            pl.BlockSpec((TILE, NUM_EXPERTS), lambda i: (i, 0)),
        ],
        out_shape=[
            jax.ShapeDtypeStruct((M, 1), jnp.int32),
            jax.ShapeDtypeStruct((M, NUM_EXPERTS), jnp.float32),
        ],
        compiler_params=pltpu.CompilerParams(
            dimension_semantics=("parallel",),
        ),
    )(xf, signatures)
    return idx.reshape(B, T), soft.reshape(B, T, NUM_EXPERTS)


# speedup vs baseline: 1.1649x; 1.1649x over previous
import jax
import jax.numpy as jnp
from jax.experimental import pallas as pl
from jax.experimental.pallas import tpu as pltpu

D_MODEL = 2048
NUM_EXPERTS = 64
MAX_SEQ_LEN = 4096
SPREAD = 2.0

TILE = 2048


def _router_kernel(x_ref, sig_ref, idx_ref, soft_ref):
    i = pl.program_id(0)
    x = x_ref[...]
    sigs = jnp.sign(sig_ref[...])
    scores = jax.lax.dot_general(
        x, sigs, (((1,), (1,)), ((), ())),
        preferred_element_type=jnp.float32)

    t0 = (i * TILE) % MAX_SEQ_LEN
    t = t0 + jax.lax.broadcasted_iota(
        jnp.int32, (TILE, NUM_EXPERTS), 0).astype(jnp.float32)
    centers = jax.lax.broadcasted_iota(
        jnp.int32, (TILE, NUM_EXPERTS), 1).astype(jnp.float32)
    u = (t * (NUM_EXPERTS / MAX_SEQ_LEN) - centers) * (1.0 / SPREAD)
    a = jnp.abs(u)
    pos = jnp.where(
        a < 1.0, 2.0 / 3.0 - a * a + 0.5 * a * a * a,
        jnp.where(a < 2.0, (1.0 / 6.0) * (2.0 - a) ** 3, 0.0))

    combined = scores * pos

    m = jnp.max(combined, axis=1, keepdims=True)
    e = jnp.exp(combined - m)
    s = jnp.sum(e, axis=1, keepdims=True)
    soft_ref[...] = e / s

    lane = jax.lax.broadcasted_iota(jnp.int32, (TILE, NUM_EXPERTS), 1)
    cand = jnp.where(combined == m, lane, NUM_EXPERTS)
    idx_ref[...] = jnp.min(cand, axis=1, keepdims=True)


def kernel(x, signatures):
    B, T, D = x.shape
    M = B * T
    xf = x.reshape(M, D)
    grid = (M // TILE,)
    idx, soft = pl.pallas_call(
        _router_kernel,
        grid=grid,
        in_specs=[
            pl.BlockSpec((TILE, D), lambda i: (i, 0)),
            pl.BlockSpec((NUM_EXPERTS, D), lambda i: (0, 0)),
        ],
        out_specs=[
            pl.BlockSpec((TILE, 1), lambda i: (i, 0)),
            pl.BlockSpec((TILE, NUM_EXPERTS), lambda i: (i, 0)),
        ],
        out_shape=[
            jax.ShapeDtypeStruct((M, 1), jnp.int32),
            jax.ShapeDtypeStruct((M, NUM_EXPERTS), jnp.float32),
        ],
        compiler_params=pltpu.CompilerParams(
            dimension_semantics=("parallel",),
        ),
    )(xf, signatures)
    return idx.reshape(B, T), soft.reshape(B, T, NUM_EXPERTS)
